# baseline (speedup 1.0000x reference)
import functools

import jax
import jax.numpy as jnp
from jax import lax
from jax.experimental import pallas as pl
from jax.experimental.pallas import tpu as pltpu
from jax.experimental.pallas import tpu_sc as plsc

_NUM_WORKERS = 32
_CHUNK = 15
_SLOT = 16


def _gather_rows(ids_pad, table, n, chunks):
    v_rows, d = table.shape
    rows_per_worker = n // _NUM_WORKERS
    n_slots = ids_pad.shape[0] // _NUM_WORKERS
    n_chunks = len(chunks)

    mesh = plsc.VectorSubcoreMesh(core_axis_name="c", subcore_axis_name="s")
    num_cores = mesh.num_cores

    @functools.partial(
        pl.kernel,
        out_type=jax.ShapeDtypeStruct((n, d), jnp.float32),
        mesh=mesh,
        scratch_types=[
            pltpu.VMEM((n_slots,), jnp.int32),
            pltpu.VMEM((2, _CHUNK, d), jnp.float32),
            pltpu.SemaphoreType.DMA,
            pltpu.SemaphoreType.DMA,
            pltpu.SemaphoreType.DMA,
            pltpu.SemaphoreType.DMA,
        ],
        compiler_params=pltpu.CompilerParams(use_tc_tiling_on_sc=False),
    )
    def body(ids_hbm, table_hbm, out_hbm, idx_v, bufs, g0, g1, s0, s1):
        gsem = (g0, g1)
        ssem = (s0, s1)
        wid = lax.axis_index("s") * num_cores + lax.axis_index("c")
        base = wid * rows_per_worker
        pltpu.sync_copy(ids_hbm.at[pl.ds(wid * n_slots, n_slots)], idx_v)

        def start_gather(ci, p):
            _, l_i = chunks[ci]
            pltpu.async_copy(
                table_hbm.at[idx_v.at[pl.ds(ci * _SLOT, l_i)]],
                bufs.at[p].at[pl.ds(0, l_i)], gsem[p])

        def start_store(ci, p):
            s_i, l_i = chunks[ci]
            pltpu.async_copy(
                bufs.at[p].at[pl.ds(0, l_i)],
                out_hbm.at[pl.ds(base + s_i, l_i)], ssem[p])

        def wait_gather(ci, p):
            _, l_i = chunks[ci]
            pltpu.make_async_copy(
                table_hbm.at[idx_v.at[pl.ds(0, l_i)]],
                bufs.at[p].at[pl.ds(0, l_i)], gsem[p]).wait()

        def wait_store(ci, p):
            _, l_i = chunks[ci]
            pltpu.make_async_copy(
                bufs.at[p].at[pl.ds(0, l_i)],
                out_hbm.at[pl.ds(base, l_i)], ssem[p]).wait()

        start_gather(0, 0)
        for ci in range(n_chunks):
            p = ci % 2
            wait_gather(ci, p)
            if ci >= 1:
                wait_store(ci - 1, 1 - p)
            if ci + 1 < n_chunks:
                start_gather(ci + 1, 1 - p)
            start_store(ci, p)
        wait_store(n_chunks - 1, (n_chunks - 1) % 2)

    return body(ids_pad, table)


def kernel(input_ids, table):
    b, s = input_ids.shape
    d = table.shape[1]
    n = b * s
    rows_per_worker = n // _NUM_WORKERS

    chunks = []
    pos = 0
    while pos < rows_per_worker:
        l_i = min(_CHUNK, rows_per_worker - pos)
        chunks.append((pos, l_i))
        pos += l_i
    n_slots = len(chunks) * _SLOT

    ids_w = input_ids.reshape(_NUM_WORKERS, rows_per_worker)
    ids_pad = jnp.zeros((_NUM_WORKERS, n_slots), dtype=jnp.int32)
    for ci, (s_i, l_i) in enumerate(chunks):
        ids_pad = lax.dynamic_update_slice(
            ids_pad, lax.dynamic_slice(ids_w, (0, s_i), (_NUM_WORKERS, l_i)),
            (0, ci * _SLOT))

    out = _gather_rows(ids_pad.reshape(-1), table, n, chunks)
    return out.reshape(b, s, d)


# speedup vs baseline: 2.6696x; 2.6696x over previous
import functools

import jax
import jax.numpy as jnp
from jax import lax
from jax.experimental import pallas as pl
from jax.experimental.pallas import tpu as pltpu
from jax.experimental.pallas import tpu_sc as plsc

_NUM_WORKERS = 32
_CHUNK = 8


def _gather_rows(ids_flat, table):
    n = ids_flat.shape[0]
    v_rows, d = table.shape
    rows_per_worker = n // _NUM_WORKERS
    n_chunks = rows_per_worker // _CHUNK
    n_main = (n_chunks - 2) // 3 * 3
    assert n_chunks - n_main == 2

    mesh = plsc.VectorSubcoreMesh(core_axis_name="c", subcore_axis_name="s")
    num_cores = mesh.num_cores

    @functools.partial(
        pl.kernel,
        out_type=jax.ShapeDtypeStruct((n, d), jnp.float32),
        mesh=mesh,
        scratch_types=[
            pltpu.VMEM((rows_per_worker,), jnp.int32),
            pltpu.VMEM((3, _CHUNK, d), jnp.float32),
            pltpu.SemaphoreType.DMA,
            pltpu.SemaphoreType.DMA,
            pltpu.SemaphoreType.DMA,
            pltpu.SemaphoreType.DMA,
            pltpu.SemaphoreType.DMA,
            pltpu.SemaphoreType.DMA,
        ],
    )
    def body(ids_hbm, table_hbm, out_hbm, idx_v, bufs, g0, g1, g2, s0, s1, s2):
        gsem = (g0, g1, g2)
        ssem = (s0, s1, s2)
        wid = lax.axis_index("s") * num_cores + lax.axis_index("c")
        base = wid * rows_per_worker
        pltpu.sync_copy(ids_hbm.at[pl.ds(base, rows_per_worker)], idx_v)

        def start_gather(chunk, p):
            off = pl.multiple_of(chunk * _CHUNK, 8)
            pltpu.async_copy(
                table_hbm.at[idx_v.at[pl.ds(off, _CHUNK)]], bufs.at[p], gsem[p])

        def start_store(chunk, p):
            pltpu.async_copy(
                bufs.at[p], out_hbm.at[pl.ds(base + chunk * _CHUNK, _CHUNK)],
                ssem[p])

        def wait_gather(p):
            pltpu.make_async_copy(
                table_hbm.at[idx_v.at[pl.ds(0, _CHUNK)]], bufs.at[p],
                gsem[p]).wait()

        def wait_store(p):
            pltpu.make_async_copy(
                bufs.at[p], out_hbm.at[pl.ds(base, _CHUNK)], ssem[p]).wait()

        start_gather(0, 0)
        start_gather(1, 1)

        @pl.loop(0, n_main, step=3)
        def _chunk_loop(g):
            for p in (0, 1, 2):
                cur = g + p

                @pl.when(cur >= 1)
                def _():
                    wait_store((p + 2) % 3)

                start_gather(cur + 2, (p + 2) % 3)
                wait_gather(p)
                start_store(cur, p)

        for cur in (n_main, n_main + 1):
            p = cur % 3
            wait_gather(p)
            start_store(cur, p)
        for cur in (n_chunks - 3, n_chunks - 2, n_chunks - 1):
            wait_store(cur % 3)

    return body(ids_flat, table)


def kernel(input_ids, table):
    b, s = input_ids.shape
    d = table.shape[1]
    out = _gather_rows(input_ids.reshape(b * s), table)
    return out.reshape(b, s, d)
